# baseline (speedup 1.0000x reference)
import functools

import jax
import jax.numpy as jnp
from jax import lax
from jax.experimental import pallas as pl
from jax.experimental.pallas import tpu as pltpu
from jax.experimental.pallas import tpu_sc as plsc

D = 64
DP = 128
B = 1024
T = 512
NC = 2
NS = 16
NW = NC * NS
N = B * T
R_PER_W = N // NW
CH = 256
NCHUNK = R_PER_W // CH
NBUF = 2
TP = T // CH
LANES = 16


def _emb_kernel(idx_hbm, table_hbm, pos_hbm, out_hbm,
                idx_v, pos_t, g0, g1, x0, x1, gsem, osem):
    g = (g0, g1)
    x = (x0, x1)
    wid = lax.axis_index("s") * NC + lax.axis_index("c")
    base = wid * R_PER_W
    iota = jnp.arange(LANES, dtype=jnp.int32)

    pltpu.sync_copy(idx_hbm.at[pl.ds(base, R_PER_W)], idx_v)

    def xpose_add(src, dst, dst_t0, pos_t0):
        def col_body(c, carry):
            idxc = jnp.full((LANES,), c, jnp.int32)
            for r0 in range(0, CH, LANES):
                v = plsc.load_gather(src, [r0 + iota, idxc])
                if pos_t0 is not None:
                    v = v + pos_t[c, pl.ds(pos_t0 + r0, LANES)]
                dst[c, pl.ds(dst_t0 + r0, LANES)] = v
            return carry

        lax.fori_loop(0, D, col_body, 0)

    for pc in range(TP):
        pltpu.sync_copy(pos_hbm.at[pl.ds(pc * CH, CH), pl.ds(0, D)], g0)
        xpose_add(g0, pos_t, pc * CH, None)

    def issue(i, j):
        pltpu.async_copy(
            table_hbm.at[idx_v.at[pl.ds(i * CH, CH)]], g[j], gsem.at[j]
        )

    def wait_gather(i, j):
        pltpu.make_async_copy(
            table_hbm.at[idx_v.at[pl.ds(i * CH, CH)]], g[j], gsem.at[j]
        ).wait()

    def out_slice(i):
        tok0 = base + i * CH
        return out_hbm.at[tok0 // T, :, pl.ds(tok0 % T, CH)]

    def start_out(i, j):
        pltpu.async_copy(x[j], out_slice(i), osem.at[j])

    def wait_out(i, j):
        pltpu.make_async_copy(x[j], out_slice(i), osem.at[j]).wait()

    issue(0, 0)

    def group(gi, carry):
        for j in range(NBUF):
            i = gi * NBUF + j
            j2 = (j + 1) % NBUF

            @pl.when(i + 1 < NCHUNK)
            def _issue_ahead():
                issue(i + 1, j2)

            wait_gather(i, j)

            @pl.when(i >= NBUF)
            def _wait_x_free():
                wait_out(i - NBUF, j)

            xpose_add(g[j], x[j], 0, (i % TP) * CH)
            start_out(i, j)
        return carry

    lax.fori_loop(0, NCHUNK // NBUF, group, 0)

    for j in range(NBUF):
        wait_out(NCHUNK - NBUF + j, j)


def _flatten_idx_tc(x):
    blk = 64

    def body(x_ref, o_ref):
        o_ref[...] = x_ref[...].reshape(blk * T)

    return pl.pallas_call(
        body,
        grid=(B // blk,),
        in_specs=[pl.BlockSpec((blk, T), lambda i: (i, 0))],
        out_specs=pl.BlockSpec((blk * T,), lambda i: (i,)),
        out_shape=jax.ShapeDtypeStruct((N,), jnp.int32),
    )(x)


def kernel(inputs, table, pos):
    idx = _flatten_idx_tc(inputs.astype(jnp.int32))
    pos2d = jnp.pad(pos.reshape(T, D).astype(jnp.float32),
                    ((0, 0), (0, DP - D)))

    mesh = plsc.VectorSubcoreMesh(core_axis_name="c", subcore_axis_name="s")
    run = functools.partial(
        pl.kernel,
        mesh=mesh,
        compiler_params=pltpu.CompilerParams(
            use_tc_tiling_on_sc=False, needs_layout_passes=False
        ),
        out_type=jax.ShapeDtypeStruct((B, D, T), jnp.float32),
        scratch_types=[
            pltpu.VMEM((R_PER_W,), jnp.int32),
            pltpu.VMEM((D, T), jnp.float32),
            pltpu.VMEM((CH, D), jnp.float32),
            pltpu.VMEM((CH, D), jnp.float32),
            pltpu.VMEM((D, CH), jnp.float32),
            pltpu.VMEM((D, CH), jnp.float32),
            pltpu.SemaphoreType.DMA((NBUF,)),
            pltpu.SemaphoreType.DMA((NBUF,)),
        ],
    )(_emb_kernel)
    out_t = run(idx, table, pos2d)
    return jnp.transpose(out_t, (0, 2, 1))


# speedup vs baseline: 3.1575x; 3.1575x over previous
import functools

import jax
import jax.numpy as jnp
from jax import lax
from jax.experimental import pallas as pl
from jax.experimental.pallas import tpu as pltpu
from jax.experimental.pallas import tpu_sc as plsc

D = 64
DP = 128
B = 1024
T = 512
NC = 2
NS = 16
NW = NC * NS
N = B * T
R_PER_W = N // NW
CH = 256
NCHUNK = R_PER_W // CH
NBUF = 4
LOOK = 2
TP = T // CH
LANES = 16


def _emb_kernel(idx_hbm, table_hbm, pos_hbm, out_hbm,
                idx_v, pos_v, rows0, rows1, rows2, rows3, gsem, osem):
    rows = (rows0, rows1, rows2, rows3)
    wid = lax.axis_index("s") * NC + lax.axis_index("c")
    base = wid * R_PER_W
    pltpu.sync_copy(pos_hbm.at[:, pl.ds(0, D)], pos_v)
    pltpu.sync_copy(idx_hbm.at[pl.ds(base, R_PER_W)], idx_v)

    def issue(i, j):
        pltpu.async_copy(
            table_hbm.at[idx_v.at[pl.ds(i * CH, CH)]], rows[j], gsem.at[j]
        )

    def wait_gather(i, j):
        pltpu.make_async_copy(
            table_hbm.at[idx_v.at[pl.ds(i * CH, CH)]], rows[j], gsem.at[j]
        ).wait()

    def start_out(i, j):
        pltpu.async_copy(
            rows[j],
            out_hbm.at[pl.ds(base + i * CH, CH), pl.ds(0, D)],
            osem.at[j],
        )

    def wait_out(i, j):
        pltpu.make_async_copy(
            rows[j],
            out_hbm.at[pl.ds(base + i * CH, CH), pl.ds(0, D)],
            osem.at[j],
        ).wait()

    for i in range(LOOK):
        issue(i, i % NBUF)

    def group(g, carry):
        for j in range(NBUF):
            i = g * NBUF + j
            j2 = (j + LOOK) % NBUF

            @pl.when(i + LOOK < NCHUNK)
            def _issue_ahead():
                @pl.when(i + LOOK >= NBUF)
                def _wait_buf_free():
                    wait_out(i + LOOK - NBUF, j2)

                issue(i + LOOK, j2)

            wait_gather(i, j)
            po = (i % TP) * CH

            def row_body(r, c2):
                for c in range(D // LANES):
                    sl = pl.ds(c * LANES, LANES)
                    rows[j][r, sl] = rows[j][r, sl] + pos_v[po + r, sl]
                return c2

            lax.fori_loop(0, CH, row_body, 0)
            start_out(i, j)
        return carry

    lax.fori_loop(0, NCHUNK // NBUF, group, 0)

    for j in range(NBUF):
        wait_out(NCHUNK - NBUF + j, j)


def _flatten_idx_tc(x):
    blk = 64

    def body(x_ref, o_ref):
        o_ref[...] = x_ref[...].reshape(blk * T)

    return pl.pallas_call(
        body,
        grid=(B // blk,),
        in_specs=[pl.BlockSpec((blk, T), lambda i: (i, 0))],
        out_specs=pl.BlockSpec((blk * T,), lambda i: (i,)),
        out_shape=jax.ShapeDtypeStruct((N,), jnp.int32),
    )(x)


def _retile_tc(x):
    bk = 8

    def body(x_ref, o_ref):
        v = x_ref[:, :D].reshape(bk, T, D)
        o_ref[...] = jnp.transpose(v, (0, 2, 1))

    return pl.pallas_call(
        body,
        grid=(B // bk,),
        in_specs=[pl.BlockSpec((bk * T, DP), lambda i: (i, 0))],
        out_specs=pl.BlockSpec((bk, D, T), lambda i: (i, 0, 0)),
        out_shape=jax.ShapeDtypeStruct((B, D, T), jnp.float32),
    )(x)


def kernel(inputs, table, pos):
    idx = _flatten_idx_tc(inputs.astype(jnp.int32))
    pos2d = jnp.pad(pos.reshape(T, D).astype(jnp.float32),
                    ((0, 0), (0, DP - D)))

    mesh = plsc.VectorSubcoreMesh(core_axis_name="c", subcore_axis_name="s")
    run = functools.partial(
        pl.kernel,
        mesh=mesh,
        compiler_params=pltpu.CompilerParams(use_tc_tiling_on_sc=False),
        out_type=jax.ShapeDtypeStruct((N, DP), jnp.float32),
        scratch_types=[
            pltpu.VMEM((R_PER_W,), jnp.int32),
            pltpu.VMEM((T, D), jnp.float32),
            pltpu.VMEM((CH, D), jnp.float32),
            pltpu.VMEM((CH, D), jnp.float32),
            pltpu.VMEM((CH, D), jnp.float32),
            pltpu.VMEM((CH, D), jnp.float32),
            pltpu.SemaphoreType.DMA((NBUF,)),
            pltpu.SemaphoreType.DMA((NBUF,)),
        ],
    )(_emb_kernel)
    out128 = run(idx, table, pos2d)
    return jnp.transpose(_retile_tc(out128), (0, 2, 1))


# speedup vs baseline: 3.7706x; 1.1942x over previous
import functools

import jax
import jax.numpy as jnp
from jax import lax
from jax.experimental import pallas as pl
from jax.experimental.pallas import tpu as pltpu
from jax.experimental.pallas import tpu_sc as plsc

D = 64
DP = 128
B = 1024
T = 512
NC = 2
NS = 16
NW = NC * NS
N = B * T
R_PER_W = N // NW
CH = 256
NCHUNK = R_PER_W // CH
NBUF = 4
LOOK = 3
TP = T // CH
LANES = 16


def _emb_kernel(idx_hbm, table_hbm, pos_hbm, out_hbm,
                idx_v, pos_v, rows0, rows1, rows2, rows3, gsem, osem):
    rows = (rows0, rows1, rows2, rows3)
    wid = lax.axis_index("s") * NC + lax.axis_index("c")
    base = wid * R_PER_W
    pltpu.sync_copy(pos_hbm.at[:, pl.ds(0, D)], pos_v)
    pltpu.sync_copy(idx_hbm.at[pl.ds(base, R_PER_W)], idx_v)

    def issue(i, j):
        pltpu.async_copy(
            table_hbm.at[idx_v.at[pl.ds(i * CH, CH)]], rows[j], gsem.at[j]
        )

    def wait_gather(i, j):
        pltpu.make_async_copy(
            table_hbm.at[idx_v.at[pl.ds(i * CH, CH)]], rows[j], gsem.at[j]
        ).wait()

    def start_out(i, j):
        pltpu.async_copy(
            rows[j],
            out_hbm.at[pl.ds(base + i * CH, CH), pl.ds(0, D)],
            osem.at[j],
        )

    def wait_out(i, j):
        pltpu.make_async_copy(
            rows[j],
            out_hbm.at[pl.ds(base + i * CH, CH), pl.ds(0, D)],
            osem.at[j],
        ).wait()

    for i in range(LOOK):
        issue(i, i % NBUF)

    def group(g, carry):
        for j in range(NBUF):
            i = g * NBUF + j
            j2 = (j + LOOK) % NBUF

            @pl.when(i + LOOK < NCHUNK)
            def _issue_ahead():
                @pl.when(i + LOOK >= NBUF)
                def _wait_buf_free():
                    wait_out(i + LOOK - NBUF, j2)

                issue(i + LOOK, j2)

            wait_gather(i, j)
            po = (i % TP) * CH

            def row_body(r, c2):
                for c in range(D // LANES):
                    sl = pl.ds(c * LANES, LANES)
                    rows[j][r, sl] = rows[j][r, sl] + pos_v[po + r, sl]
                return c2

            lax.fori_loop(0, CH, row_body, 0)
            start_out(i, j)
        return carry

    lax.fori_loop(0, NCHUNK // NBUF, group, 0)

    for j in range(NBUF):
        wait_out(NCHUNK - NBUF + j, j)


def _flatten_idx_tc(x):
    blk = 64

    def body(x_ref, o_ref):
        o_ref[...] = x_ref[...].reshape(blk * T)

    return pl.pallas_call(
        body,
        grid=(B // blk,),
        in_specs=[pl.BlockSpec((blk, T), lambda i: (i, 0))],
        out_specs=pl.BlockSpec((blk * T,), lambda i: (i,)),
        out_shape=jax.ShapeDtypeStruct((N,), jnp.int32),
    )(x)


def kernel(inputs, table, pos):
    idx = _flatten_idx_tc(inputs.astype(jnp.int32))
    pos2d = jnp.pad(pos.reshape(T, D).astype(jnp.float32),
                    ((0, 0), (0, DP - D)))

    mesh = plsc.VectorSubcoreMesh(core_axis_name="c", subcore_axis_name="s")
    run = functools.partial(
        pl.kernel,
        mesh=mesh,
        compiler_params=pltpu.CompilerParams(use_tc_tiling_on_sc=False),
        out_type=jax.ShapeDtypeStruct((N, DP), jnp.float32),
        scratch_types=[
            pltpu.VMEM((R_PER_W,), jnp.int32),
            pltpu.VMEM((T, D), jnp.float32),
            pltpu.VMEM((CH, D), jnp.float32),
            pltpu.VMEM((CH, D), jnp.float32),
            pltpu.VMEM((CH, D), jnp.float32),
            pltpu.VMEM((CH, D), jnp.float32),
            pltpu.SemaphoreType.DMA((NBUF,)),
            pltpu.SemaphoreType.DMA((NBUF,)),
        ],
    )(_emb_kernel)
    out128 = run(idx, table, pos2d)
    return out128[:, :D].reshape(B, T, D)


# speedup vs baseline: 4.1225x; 1.0933x over previous
import functools

import jax
import jax.numpy as jnp
from jax import lax
from jax.experimental import pallas as pl
from jax.experimental.pallas import tpu as pltpu
from jax.experimental.pallas import tpu_sc as plsc

D = 64
DP = 128
B = 1024
T = 512
NC = 2
NS = 16
NW = NC * NS
N = B * T
R_PER_W = N // NW
CH = 256
NCHUNK = R_PER_W // CH
NBUF = 4
LOOK = 2
TP = T // CH
LANES = 16


def _emb_kernel(idx_hbm, table_hbm, pos_hbm, out_hbm,
                idx_v, pos_v, rows0, rows1, rows2, rows3, gsem, osem):
    rows = (rows0, rows1, rows2, rows3)
    wid = lax.axis_index("s") * NC + lax.axis_index("c")
    base = wid * R_PER_W
    pltpu.sync_copy(pos_hbm.at[:, pl.ds(0, D)], pos_v)
    pltpu.sync_copy(idx_hbm.at[pl.ds(base, R_PER_W)], idx_v)

    def issue(i, j):
        pltpu.async_copy(
            table_hbm.at[idx_v.at[pl.ds(i * CH, CH)]], rows[j], gsem.at[j]
        )

    def wait_gather(i, j):
        pltpu.make_async_copy(
            table_hbm.at[idx_v.at[pl.ds(i * CH, CH)]], rows[j], gsem.at[j]
        ).wait()

    def start_out(i, j):
        pltpu.async_copy(
            rows[j],
            out_hbm.at[pl.ds(base + i * CH, CH), pl.ds(0, D)],
            osem.at[j],
        )

    def wait_out(i, j):
        pltpu.make_async_copy(
            rows[j],
            out_hbm.at[pl.ds(base + i * CH, CH), pl.ds(0, D)],
            osem.at[j],
        ).wait()

    for i in range(LOOK):
        issue(i, i % NBUF)

    def group(g, carry):
        for j in range(NBUF):
            i = g * NBUF + j
            j2 = (j + LOOK) % NBUF

            @pl.when(i + LOOK < NCHUNK)
            def _issue_ahead():
                @pl.when(i + LOOK >= NBUF)
                def _wait_buf_free():
                    wait_out(i + LOOK - NBUF, j2)

                issue(i + LOOK, j2)

            wait_gather(i, j)
            po = (i % TP) * CH

            def row_body(r, c2):
                for c in range(D // LANES):
                    sl = pl.ds(c * LANES, LANES)
                    rows[j][r, sl] = rows[j][r, sl] + pos_v[po + r, sl]
                return c2

            lax.fori_loop(0, CH, row_body, 0)
            start_out(i, j)
        return carry

    lax.fori_loop(0, NCHUNK // NBUF, group, 0)

    for j in range(NBUF):
        wait_out(NCHUNK - NBUF + j, j)


def _flatten_idx_tc(x):
    blk = 64

    def body(x_ref, o_ref):
        o_ref[...] = x_ref[...].reshape(blk * T)

    return pl.pallas_call(
        body,
        grid=(B // blk,),
        in_specs=[pl.BlockSpec((blk, T), lambda i: (i, 0))],
        out_specs=pl.BlockSpec((blk * T,), lambda i: (i,)),
        out_shape=jax.ShapeDtypeStruct((N,), jnp.int32),
    )(x)


def kernel(inputs, table, pos):
    idx = _flatten_idx_tc(inputs.astype(jnp.int32))
    pos2d = jnp.pad(pos.reshape(T, D).astype(jnp.float32),
                    ((0, 0), (0, DP - D)))

    mesh = plsc.VectorSubcoreMesh(core_axis_name="c", subcore_axis_name="s")
    run = functools.partial(
        pl.kernel,
        mesh=mesh,
        compiler_params=pltpu.CompilerParams(use_tc_tiling_on_sc=False),
        out_type=jax.ShapeDtypeStruct((N, DP), jnp.float32),
        scratch_types=[
            pltpu.VMEM((R_PER_W,), jnp.int32),
            pltpu.VMEM((T, D), jnp.float32),
            pltpu.VMEM((CH, D), jnp.float32),
            pltpu.VMEM((CH, D), jnp.float32),
            pltpu.VMEM((CH, D), jnp.float32),
            pltpu.VMEM((CH, D), jnp.float32),
            pltpu.SemaphoreType.DMA((NBUF,)),
            pltpu.SemaphoreType.DMA((NBUF,)),
        ],
    )(_emb_kernel)
    out128 = run(idx, table, pos2d)
    return out128[:, :D].reshape(B, T, D)
